# baseline (speedup 1.0000x reference)
import functools

import jax
import jax.numpy as jnp
from jax import lax
from jax.experimental import pallas as pl
from jax.experimental.pallas import tpu as pltpu
from jax.experimental.pallas import tpu_sc as plsc

EMB = 16
FIELDS = 26
BATCH = 16384
VOCAB_P1 = 1000001
VPAD = 1000064
B_TOTAL = BATCH * FIELDS
NC, NS = 2, 16
NW = NC * NS
NBH = BATCH // (128 * NW)
CHUNK = 128 * FIELDS

TW = 1024
TWE = TW * EMB
NFULL = VPAD // TW
EXTRA = 512
TAILC = 999936
NTAIL = VOCAB_P1 - TAILC

_mesh = plsc.VectorSubcoreMesh(core_axis_name="c", subcore_axis_name="s")


@functools.partial(
    pl.kernel,
    mesh=_mesh,
    out_type=jax.ShapeDtypeStruct((VPAD * EMB,), jnp.float32),
    scratch_types=[
        pltpu.VMEM((2, EMB, TW), jnp.float32),
        pltpu.VMEM((2 * TWE,), jnp.float32),
        pltpu.VMEM((512 * 17,), jnp.float32),
        pltpu.SemaphoreType.DMA,
        pltpu.SemaphoreType.DMA,
    ],
    compiler_params=pltpu.CompilerParams(
        use_tc_tiling_on_sc=True,
        needs_layout_passes=False,
        disable_bounds_checks=True,
    ),
)
def _table_kernel(tt_hbm, tail_hbm, out_hbm, in_v, tv, tvp, sem_i, sem_o):
    wid = lax.axis_index("s") * NC + lax.axis_index("c")
    iota16 = lax.iota(jnp.int32, 16)
    i17s = iota16 * 17

    def in_copy(bid, buf):
        pltpu.async_copy(tt_hbm.at[:, pl.ds(bid * TW, TW)], in_v.at[buf], sem_i)

    def in_wait():
        pltpu.make_async_copy(
            tt_hbm.at[:, pl.ds(0, TW)], in_v.at[0], sem_i).wait()

    HWE = TWE // 2

    def out_half_copy(bid, buf, h):
        pltpu.async_copy(
            tv.at[pl.ds(buf * TWE + h * HWE, HWE)],
            out_hbm.at[pl.ds(bid * TWE + h * HWE, HWE)], sem_o)

    def out_half_wait():
        pltpu.make_async_copy(
            tv.at[pl.ds(0, HWE)], out_hbm.at[pl.ds(0, HWE)], sem_o).wait()

    def transpose_half(buf, h, tbase):
        for jj in range(32):
            j0 = jj * 16
            for d in range(EMB):
                v = in_v[buf, d, pl.ds(h * 512 + j0, 16)]
                plsc.store_scatter(tvp, [i17s + (j0 * 17 + d)], v)
        for jj in range(32):
            for l in range(16):
                jr = jj * 16 + l
                row = plsc.load_gather(tvp, [iota16 + jr * 17])
                tv[pl.ds(tbase + (h * 512 + jr) * EMB, EMB)] = row

    nblk = 30 + (wid < 16).astype(jnp.int32)
    in_copy(wid, 0)

    def body(k2, _):
        k = k2 // 2
        h = k2 % 2
        buf = (k % 2).astype(jnp.int32)
        bid = wid + k * NW
        @pl.when(h == 0)
        def _():
            in_wait()
            @pl.when(k + 1 < nblk)
            def _():
                in_copy(bid + NW, 1 - buf)
        transpose_half(buf, h, buf * TWE)
        @pl.when(k2 > 1)
        def _():
            out_half_wait()
        out_half_copy(bid, buf, h)
        return ()

    lax.fori_loop(0, 2 * nblk, body, ())
    out_half_wait()
    out_half_wait()

    @pl.when(wid == 16)
    def _():
        pltpu.async_copy(
            tt_hbm.at[:, pl.ds(NFULL * TW, EXTRA)],
            in_v.at[0, :, pl.ds(0, EXTRA)], sem_i).wait()
        transpose_half(0, 0, 0)
        pltpu.async_copy(
            tv.at[pl.ds(0, EXTRA * EMB)],
            out_hbm.at[pl.ds(NFULL * TW * EMB, EXTRA * EMB)], sem_o).wait()

    @pl.when(wid == 17)
    def _():
        pltpu.async_copy(tail_hbm, tv.at[pl.ds(0, 1152)], sem_i).wait()
        pltpu.async_copy(
            tv.at[pl.ds(0, 1152)],
            out_hbm.at[pl.ds(TAILC * EMB, 1152)], sem_o).wait()


@functools.partial(
    pl.kernel,
    mesh=_mesh,
    out_type=jax.ShapeDtypeStruct((FIELDS, 2, BATCH // 128, 8, 128), jnp.float32),
    scratch_types=[
        pltpu.VMEM((2, CHUNK), jnp.int32),
        pltpu.VMEM((2, CHUNK, EMB), jnp.float32),
        pltpu.VMEM((4 * EMB, 133), jnp.float32),
        pltpu.SemaphoreType.DMA,
        pltpu.SemaphoreType.DMA,
        pltpu.SemaphoreType.DMA,
    ],
    compiler_params=pltpu.CompilerParams(
        use_tc_tiling_on_sc=False, needs_layout_passes=False),
)
def _emb_kernel(idx_hbm, table_hbm, out_hbm, idx_v, rows_v, tr_v, sem_i, sem_g, sem_o):
    wid = lax.axis_index("s") * NC + lax.axis_index("c")
    iota16 = lax.iota(jnp.int32, 16)

    def idx_copy(c):
        bh = wid * NBH + c
        return pltpu.async_copy(
            idx_hbm.at[pl.ds(bh * CHUNK, CHUNK)], idx_v.at[c % 2], sem_i)

    def gather(c):
        return pltpu.async_copy(
            table_hbm.at[idx_v.at[c % 2]], rows_v.at[c % 2], sem_g)

    def out_wait(out_ref):
        pltpu.make_async_copy(
            tr_v.at[pl.ds(0, 8), pl.ds(0, 128)], out_ref.at[0, 0, 0], sem_o).wait()

    def transpose_and_store(c):
        bh = wid * NBH + c
        rbuf = c % 2

        def fbody(f, _):
            tq = (f % 4).astype(jnp.int32) * EMB
            d_vec = tq + iota16
            @pl.when(f >= 4)
            def _():
                out_wait(out_hbm)
                out_wait(out_hbm)
            for bl in range(128):
                v = rows_v[rbuf, bl * FIELDS + f, :]
                plsc.store_scatter(tr_v, [d_vec, iota16 * 0 + bl], v)
            for dh in range(2):
                pltpu.async_copy(
                    tr_v.at[pl.ds(tq + dh * 8, 8), pl.ds(0, 128)],
                    out_hbm.at[f, dh, bh], sem_o)
            return ()

        lax.fori_loop(0, FIELDS, fbody, ())
        for _ in range(8):
            out_wait(out_hbm)

    idx_copy(0).wait()
    g = gather(0)
    i_next = idx_copy(1)
    for c in range(NBH):
        g.wait()
        if c + 1 < NBH:
            i_next.wait()
            g = gather(c + 1)
        if c + 2 < NBH:
            i_next = idx_copy(c + 2)
        transpose_and_store(c)


def kernel(x, table):
    idx = x.reshape(-1)
    tt = jnp.swapaxes(table, 0, 1)
    tail = jnp.pad(
        lax.slice(table, (TAILC, 0), (VOCAB_P1, EMB)),
        ((0, 72 - NTAIL), (0, 0))).reshape(-1)
    tflat = _table_kernel(tt, tail)
    tlin = tflat.reshape(VPAD, EMB)
    out5 = _emb_kernel(idx, tlin)
    return out5.transpose(2, 4, 0, 1, 3).reshape(BATCH, FIELDS, EMB)


# speedup vs baseline: 1.8179x; 1.8179x over previous
import functools

import jax
import jax.numpy as jnp
from jax import lax
from jax.experimental import pallas as pl
from jax.experimental.pallas import tpu as pltpu
from jax.experimental.pallas import tpu_sc as plsc

EMB = 16
FIELDS = 26
BATCH = 16384
VOCAB_P1 = 1000001
VPAD = 1000064
B_TOTAL = BATCH * FIELDS
NC, NS = 2, 16
NW = NC * NS
NBH = BATCH // (128 * NW)
CHUNK = 128 * FIELDS

TW = 1024
TWE = TW * EMB
NFULL = VPAD // TW
EXTRA = 512
TAILC = 999936
NTAIL = VOCAB_P1 - TAILC

_mesh = plsc.VectorSubcoreMesh(core_axis_name="c", subcore_axis_name="s")


@functools.partial(
    pl.kernel,
    mesh=_mesh,
    out_type=jax.ShapeDtypeStruct((VPAD * EMB,), jnp.float32),
    scratch_types=[
        pltpu.VMEM((2, EMB, TW), jnp.float32),
        pltpu.VMEM((2 * TWE,), jnp.float32),
        pltpu.SemaphoreType.DMA,
        pltpu.SemaphoreType.DMA,
    ],
    compiler_params=pltpu.CompilerParams(
        use_tc_tiling_on_sc=True,
        needs_layout_passes=False,
        disable_bounds_checks=True,
    ),
)
def _table_kernel(tt_hbm, tail_hbm, out_hbm, in_v, tv, sem_i, sem_o):
    wid = lax.axis_index("s") * NC + lax.axis_index("c")
    i16s = lax.iota(jnp.int32, 16) * EMB

    def in_copy(bid, buf):
        pltpu.async_copy(tt_hbm.at[:, pl.ds(bid * TW, TW)], in_v.at[buf], sem_i)

    def in_wait():
        pltpu.make_async_copy(
            tt_hbm.at[:, pl.ds(0, TW)], in_v.at[0], sem_i).wait()

    def out_copy(bid, buf):
        pltpu.async_copy(
            tv.at[pl.ds(buf * TWE, TWE)],
            out_hbm.at[pl.ds(bid * TWE, TWE)], sem_o)

    def out_wait():
        pltpu.make_async_copy(
            tv.at[pl.ds(0, TWE)], out_hbm.at[pl.ds(0, TWE)], sem_o).wait()

    def transpose_block(buf, tbase, width):
        for tc in range(width // 128):
            for jj in range(8):
                j0 = tc * 128 + jj * 16
                for d in range(EMB):
                    v = in_v[buf, d, pl.ds(j0, 16)]
                    plsc.store_scatter(tv, [i16s + (tbase + j0 * EMB + d)], v)

    nblk = 30 + (wid < 16).astype(jnp.int32)
    in_copy(wid, 0)

    def body(k, _):
        buf = (k % 2).astype(jnp.int32)
        bid = wid + k * NW
        in_wait()
        @pl.when(k + 1 < nblk)
        def _():
            in_copy(bid + NW, 1 - buf)
        transpose_block(buf, buf * TWE, TW)
        @pl.when(k > 0)
        def _():
            out_wait()
        out_copy(bid, buf)
        return ()

    lax.fori_loop(0, nblk, body, ())
    out_wait()

    @pl.when(wid == 16)
    def _():
        pltpu.async_copy(
            tt_hbm.at[:, pl.ds(NFULL * TW, EXTRA)],
            in_v.at[0, :, pl.ds(0, EXTRA)], sem_i).wait()
        transpose_block(0, 0, EXTRA)
        pltpu.async_copy(
            tv.at[pl.ds(0, EXTRA * EMB)],
            out_hbm.at[pl.ds(NFULL * TW * EMB, EXTRA * EMB)], sem_o).wait()

    @pl.when(wid == 17)
    def _():
        pltpu.async_copy(tail_hbm, tv.at[pl.ds(0, 1152)], sem_i).wait()
        pltpu.async_copy(
            tv.at[pl.ds(0, 1152)],
            out_hbm.at[pl.ds(TAILC * EMB, 1152)], sem_o).wait()


@functools.partial(
    pl.kernel,
    mesh=_mesh,
    out_type=jax.ShapeDtypeStruct((FIELDS, 2, BATCH // 128, 8, 128), jnp.float32),
    scratch_types=[
        pltpu.VMEM((2, CHUNK), jnp.int32),
        pltpu.VMEM((2, CHUNK, EMB), jnp.float32),
        pltpu.VMEM((4 * EMB, 133), jnp.float32),
        pltpu.SemaphoreType.DMA,
        pltpu.SemaphoreType.DMA,
        pltpu.SemaphoreType.DMA,
    ],
    compiler_params=pltpu.CompilerParams(
        use_tc_tiling_on_sc=False, needs_layout_passes=False),
)
def _emb_kernel(idx_hbm, table_hbm, out_hbm, idx_v, rows_v, tr_v, sem_i, sem_g, sem_o):
    wid = lax.axis_index("s") * NC + lax.axis_index("c")
    iota16 = lax.iota(jnp.int32, 16)

    def idx_copy(c):
        bh = wid * NBH + c
        return pltpu.async_copy(
            idx_hbm.at[pl.ds(bh * CHUNK, CHUNK)], idx_v.at[c % 2], sem_i)

    def gather(c):
        return pltpu.async_copy(
            table_hbm.at[idx_v.at[c % 2]], rows_v.at[c % 2], sem_g)

    def out_wait(out_ref):
        pltpu.make_async_copy(
            tr_v.at[pl.ds(0, 8), pl.ds(0, 128)], out_ref.at[0, 0, 0], sem_o).wait()

    def transpose_and_store(c):
        bh = wid * NBH + c
        rbuf = c % 2

        def fbody(f, _):
            tq = (f % 4).astype(jnp.int32) * EMB
            d_vec = tq + iota16
            @pl.when(f >= 4)
            def _():
                out_wait(out_hbm)
                out_wait(out_hbm)
            for bl in range(128):
                v = rows_v[rbuf, bl * FIELDS + f, :]
                plsc.store_scatter(tr_v, [d_vec, iota16 * 0 + bl], v)
            for dh in range(2):
                pltpu.async_copy(
                    tr_v.at[pl.ds(tq + dh * 8, 8), pl.ds(0, 128)],
                    out_hbm.at[f, dh, bh], sem_o)
            return ()

        lax.fori_loop(0, FIELDS, fbody, ())
        for _ in range(8):
            out_wait(out_hbm)

    idx_copy(0).wait()
    g = gather(0)
    i_next = idx_copy(1)
    for c in range(NBH):
        g.wait()
        if c + 1 < NBH:
            i_next.wait()
            g = gather(c + 1)
        if c + 2 < NBH:
            i_next = idx_copy(c + 2)
        transpose_and_store(c)


def kernel(x, table):
    idx = x.reshape(-1)
    tt = jnp.swapaxes(table, 0, 1)
    tail = jnp.pad(
        lax.slice(table, (TAILC, 0), (VOCAB_P1, EMB)),
        ((0, 72 - NTAIL), (0, 0))).reshape(-1)
    tflat = _table_kernel(tt, tail)
    tlin = tflat.reshape(VPAD, EMB)
    out5 = _emb_kernel(idx, tlin)
    return out5.transpose(2, 4, 0, 1, 3).reshape(BATCH, FIELDS, EMB)
